# baseline (speedup 1.0000x reference)
import functools

import jax
import jax.numpy as jnp
from jax import lax
from jax.experimental import pallas as pl
from jax.experimental.pallas import tpu as pltpu
from jax.experimental.pallas import tpu_sc as plsc

_B = 16384
_NCTX = 5
_D = 32
_LANES = 16
_VOCAB = 1000000

_BAND = 125184
_NBAND = 8
_ROWS2 = _BAND * _NBAND

_BNR = 768
_TGRID = _BAND // _BNR

_info = plsc.get_sparse_core_info()
_NC, _NS = _info.num_cores, _info.num_subcores
_NW = _NC * _NS
_BPW = _B // _NW
_CPW = _BPW * _NCTX
_CHUNK = 128


def _transpose_body(*refs):
    lo_ref, hi_ref = refs[-2], refs[-1]
    eye = jnp.eye(_D, dtype=jnp.float32)
    lo_cols, hi_cols = [], []
    for ref in refs[:-2]:
        t = jax.lax.dot_general(ref[...], eye, (((0,), (0,)), ((), ())),
                                preferred_element_type=jnp.float32,
                                precision=jax.lax.Precision.HIGHEST)
        lo_cols.append(t[:, :_LANES])
        hi_cols.append(t[:, _LANES:])
    lo_ref[...] = jnp.concatenate(lo_cols, axis=1)
    hi_ref[...] = jnp.concatenate(hi_cols, axis=1)


def _to_banded_rows(w_t):
    last_blk = (_VOCAB - 1) // _BNR
    specs = []
    for qq in range(_NBAND):
        specs.append(pl.BlockSpec(
            (_D, _BNR),
            lambda i, q=qq: (0, jnp.minimum(q * _TGRID + i, last_blk))))
    return pl.pallas_call(
        _transpose_body,
        grid=(_TGRID,),
        in_specs=specs,
        out_specs=[pl.BlockSpec((_BNR, 128), lambda i: (i, 0)),
                   pl.BlockSpec((_BNR, 128), lambda i: (i, 0))],
        out_shape=[jax.ShapeDtypeStruct((_BAND, 128), jnp.float32),
                   jax.ShapeDtypeStruct((_BAND, 128), jnp.float32)],
    )(*([w_t] * _NBAND))


_mesh = plsc.VectorSubcoreMesh(core_axis_name="c", subcore_axis_name="s")


@functools.partial(
    pl.kernel,
    mesh=_mesh,
    out_type=jax.ShapeDtypeStruct((_B * _NCTX,), jnp.float32),
    scratch_types=[
        pltpu.VMEM((_BPW,), jnp.int32),
        pltpu.VMEM((_CPW,), jnp.int32),
        pltpu.VMEM((_BPW, _LANES), jnp.float32),
        pltpu.VMEM((_BPW, _LANES), jnp.float32),
        pltpu.VMEM((_CPW, _LANES), jnp.float32),
        pltpu.VMEM((_CPW, _LANES), jnp.float32),
        pltpu.VMEM((_CPW,), jnp.float32),
        pltpu.SemaphoreType.DMA,
    ],
    compiler_params=pltpu.CompilerParams(
        needs_layout_passes=False, use_tc_tiling_on_sc=False),
)
def _act2vec_sc(t_hbm, c_hbm, wtlo_hbm, wthi_hbm, wclo_hbm, wchi_hbm, out_hbm,
                tix_v, cix_v, welo_v, wehi_v, celo_v, cehi_v, out_v, sem):
    wid = lax.axis_index("s") * _NC + lax.axis_index("c")
    tb = wid * _BPW
    cb = wid * _CPW

    pltpu.sync_copy(t_hbm.at[pl.ds(tb, _BPW)], tix_v)
    pltpu.sync_copy(c_hbm.at[pl.ds(cb, _CPW)], cix_v)

    def band_fix(ref, nvec):
        def vfix(i, carry):
            r = ref[pl.ds(i * _LANES, _LANES)]
            q = (r >= _BAND).astype(jnp.int32)
            for k in range(2, _NBAND):
                q = q + (r >= k * _BAND).astype(jnp.int32)
            ref[pl.ds(i * _LANES, _LANES)] = (r - q * _BAND) * _NBAND + q
            return carry
        lax.fori_loop(0, nvec, vfix, 0)

    band_fix(tix_v, _BPW // _LANES)
    band_fix(cix_v, _CPW // _LANES)

    copies = []
    for j in range(_BPW // _CHUNK):
        ds = pl.ds(j * _CHUNK, _CHUNK)
        copies.append(pltpu.async_copy(wtlo_hbm.at[tix_v.at[ds]],
                                       welo_v.at[ds], sem))
        copies.append(pltpu.async_copy(wthi_hbm.at[tix_v.at[ds]],
                                       wehi_v.at[ds], sem))
    for j in range(_CPW // _CHUNK):
        ds = pl.ds(j * _CHUNK, _CHUNK)
        copies.append(pltpu.async_copy(wclo_hbm.at[cix_v.at[ds]],
                                       celo_v.at[ds], sem))
        copies.append(pltpu.async_copy(wchi_hbm.at[cix_v.at[ds]],
                                       cehi_v.at[ds], sem))
    for c in copies:
        c.wait()

    last = lax.iota(jnp.int32, _LANES) == (_LANES - 1)

    def body(b, carry):
        wlo = welo_v[b, pl.ds(0, _LANES)]
        whi = wehi_v[b, pl.ds(0, _LANES)]
        for n in range(_NCTX):
            r = b * _NCTX + n
            clo = celo_v[r, pl.ds(0, _LANES)]
            chi = cehi_v[r, pl.ds(0, _LANES)]
            p = clo * wlo + chi * whi
            s = jnp.full((_LANES,), jnp.sum(p))
            plsc.store_scatter(out_v, [jnp.full((_LANES,), r, jnp.int32)],
                               s, mask=last)
        return carry

    lax.fori_loop(0, _BPW, body, 0)

    pltpu.sync_copy(out_v, out_hbm.at[pl.ds(cb, _CPW)])


def kernel(target, context, W_target, W_context):
    tflat = target.reshape(-1)
    cflat = context.reshape(-1)
    wt_lo, wt_hi = _to_banded_rows(W_target.T)
    wc_lo, wc_hi = _to_banded_rows(W_context.T)
    out = _act2vec_sc(tflat, cflat,
                      wt_lo.reshape(_ROWS2, _LANES),
                      wt_hi.reshape(_ROWS2, _LANES),
                      wc_lo.reshape(_ROWS2, _LANES),
                      wc_hi.reshape(_ROWS2, _LANES))
    return out.reshape(_B, _NCTX)


# speedup vs baseline: 1.2801x; 1.2801x over previous
import functools

import jax
import jax.numpy as jnp
from jax import lax
from jax.experimental import pallas as pl
from jax.experimental.pallas import tpu as pltpu
from jax.experimental.pallas import tpu_sc as plsc

_B = 16384
_NCTX = 5
_D = 32
_LANES = 16

_info = plsc.get_sparse_core_info()
_NC, _NS = _info.num_cores, _info.num_subcores
_NW = _NC * _NS
_BPW = _B // _NW
_CPW = _BPW * _NCTX
_CHUNK = 128

_mesh = plsc.VectorSubcoreMesh(core_axis_name="c", subcore_axis_name="s")


@functools.partial(
    pl.kernel,
    mesh=_mesh,
    out_type=jax.ShapeDtypeStruct((_B * _NCTX,), jnp.float32),
    scratch_types=[
        pltpu.VMEM((_BPW,), jnp.int32),
        pltpu.VMEM((_CPW,), jnp.int32),
        pltpu.VMEM((_BPW, _D), jnp.float32),
        pltpu.VMEM((_CPW, _D), jnp.float32),
        pltpu.VMEM((_CPW,), jnp.float32),
        pltpu.SemaphoreType.DMA,
    ],
    compiler_params=pltpu.CompilerParams(
        needs_layout_passes=False, use_tc_tiling_on_sc=False),
)
def _act2vec_sc(t_hbm, c_hbm, wt_hbm, wc_hbm, out_hbm,
                tix_v, cix_v, we_v, ce_v, out_v, sem):
    wid = lax.axis_index("s") * _NC + lax.axis_index("c")
    tb = wid * _BPW
    cb = wid * _CPW

    pltpu.sync_copy(t_hbm.at[pl.ds(tb, _BPW)], tix_v)
    pltpu.sync_copy(c_hbm.at[pl.ds(cb, _CPW)], cix_v)

    copies = []
    for j in range(_BPW // _CHUNK):
        copies.append(pltpu.async_copy(
            wt_hbm.at[tix_v.at[pl.ds(j * _CHUNK, _CHUNK)]],
            we_v.at[pl.ds(j * _CHUNK, _CHUNK)], sem))
    for j in range(_CPW // _CHUNK):
        copies.append(pltpu.async_copy(
            wc_hbm.at[cix_v.at[pl.ds(j * _CHUNK, _CHUNK)]],
            ce_v.at[pl.ds(j * _CHUNK, _CHUNK)], sem))
    for c in copies:
        c.wait()

    last = lax.iota(jnp.int32, _LANES) == (_LANES - 1)

    def body(b, carry):
        we0 = we_v[b, pl.ds(0, _LANES)]
        we1 = we_v[b, pl.ds(_LANES, _LANES)]
        for n in range(_NCTX):
            r = b * _NCTX + n
            ce0 = ce_v[r, pl.ds(0, _LANES)]
            ce1 = ce_v[r, pl.ds(_LANES, _LANES)]
            p = ce0 * we0 + ce1 * we1
            s = jnp.full((_LANES,), jnp.sum(p))
            plsc.store_scatter(out_v, [jnp.full((_LANES,), r, jnp.int32)],
                               s, mask=last)
        return carry

    lax.fori_loop(0, _BPW, body, 0)

    pltpu.sync_copy(out_v, out_hbm.at[pl.ds(cb, _CPW)])


def kernel(target, context, W_target, W_context):
    tflat = target.reshape(-1)
    cflat = context.reshape(-1)
    out = _act2vec_sc(tflat, cflat, W_target, W_context)
    return out.reshape(_B, _NCTX)


# speedup vs baseline: 1.3015x; 1.0167x over previous
import functools

import jax
import jax.numpy as jnp
from jax import lax
from jax.experimental import pallas as pl
from jax.experimental.pallas import tpu as pltpu
from jax.experimental.pallas import tpu_sc as plsc

_B = 16384
_NCTX = 5
_D = 32
_LANES = 16

_info = plsc.get_sparse_core_info()
_NC, _NS = _info.num_cores, _info.num_subcores
_NW = _NC * _NS
_BPW = _B // _NW
_CPW = _BPW * _NCTX
_CHUNK = 128

_VOCAB = 1000000
_BAND = 125184
_NBAND = 8
_ROWS2 = _BAND * _NBAND
_BNR = 768
_TGRID = _BAND // _BNR


def _transpose_body(*refs):
    lo_ref, hi_ref = refs[-2], refs[-1]
    eye = jnp.eye(_D, dtype=jnp.float32)
    lo_cols, hi_cols = [], []
    for ref in refs[:-2]:
        t = jax.lax.dot_general(ref[...], eye, (((0,), (0,)), ((), ())),
                                preferred_element_type=jnp.float32)
        lo_cols.append(t[:, :_LANES])
        hi_cols.append(t[:, _LANES:])
    lo_ref[...] = jnp.concatenate(lo_cols, axis=1)
    hi_ref[...] = jnp.concatenate(hi_cols, axis=1)


def _to_banded_rows(w_t):
    last_blk = (_VOCAB - 1) // _BNR
    specs = []
    for qq in range(_NBAND):
        specs.append(pl.BlockSpec(
            (_D, _BNR),
            lambda i, q=qq: (0, jnp.minimum(q * _TGRID + i, last_blk))))
    return pl.pallas_call(
        _transpose_body,
        grid=(_TGRID,),
        in_specs=specs,
        out_specs=[pl.BlockSpec((_BNR, 128), lambda i: (i, 0)),
                   pl.BlockSpec((_BNR, 128), lambda i: (i, 0))],
        out_shape=[jax.ShapeDtypeStruct((_BAND, 128), jnp.float32),
                   jax.ShapeDtypeStruct((_BAND, 128), jnp.float32)],
    )(*([w_t] * _NBAND))


_mesh = plsc.VectorSubcoreMesh(core_axis_name="c", subcore_axis_name="s")


@functools.partial(
    pl.kernel,
    mesh=_mesh,
    out_type=jax.ShapeDtypeStruct((_B * _NCTX,), jnp.float32),
    scratch_types=[
        pltpu.VMEM((_BPW,), jnp.int32),
        pltpu.VMEM((_CPW,), jnp.int32),
        pltpu.VMEM((_BPW, _LANES), jnp.float32),
        pltpu.VMEM((_BPW, _LANES), jnp.float32),
        pltpu.VMEM((_CPW, _LANES), jnp.float32),
        pltpu.VMEM((_CPW, _LANES), jnp.float32),
        pltpu.VMEM((_CPW,), jnp.float32),
        pltpu.SemaphoreType.DMA,
    ],
    compiler_params=pltpu.CompilerParams(
        needs_layout_passes=False, use_tc_tiling_on_sc=False),
)
def _act2vec_sc(t_hbm, c_hbm, wtlo_hbm, wthi_hbm, wclo_hbm, wchi_hbm, out_hbm,
                tix_v, cix_v, welo_v, wehi_v, celo_v, cehi_v, out_v, sem):
    wid = lax.axis_index("s") * _NC + lax.axis_index("c")
    tb = wid * _BPW
    cb = wid * _CPW

    pltpu.sync_copy(t_hbm.at[pl.ds(tb, _BPW)], tix_v)
    pltpu.sync_copy(c_hbm.at[pl.ds(cb, _CPW)], cix_v)

    def band_fix(ref, nvec):
        def vfix(i, carry):
            r = ref[pl.ds(i * _LANES, _LANES)]
            q = (r >= _BAND).astype(jnp.int32)
            for k in range(2, _NBAND):
                q = q + (r >= k * _BAND).astype(jnp.int32)
            ref[pl.ds(i * _LANES, _LANES)] = (r - q * _BAND) * _NBAND + q
            return carry
        lax.fori_loop(0, nvec, vfix, 0)

    band_fix(tix_v, _BPW // _LANES)
    band_fix(cix_v, _CPW // _LANES)

    copies = []
    for j in range(_BPW // _CHUNK):
        ds = pl.ds(j * _CHUNK, _CHUNK)
        copies.append(pltpu.async_copy(wtlo_hbm.at[tix_v.at[ds]],
                                       welo_v.at[ds], sem))
        copies.append(pltpu.async_copy(wthi_hbm.at[tix_v.at[ds]],
                                       wehi_v.at[ds], sem))
    for j in range(_CPW // _CHUNK):
        ds = pl.ds(j * _CHUNK, _CHUNK)
        copies.append(pltpu.async_copy(wclo_hbm.at[cix_v.at[ds]],
                                       celo_v.at[ds], sem))
        copies.append(pltpu.async_copy(wchi_hbm.at[cix_v.at[ds]],
                                       cehi_v.at[ds], sem))
    for c in copies:
        c.wait()

    last = lax.iota(jnp.int32, _LANES) == (_LANES - 1)

    def body(b, carry):
        wlo = welo_v[b, pl.ds(0, _LANES)]
        whi = wehi_v[b, pl.ds(0, _LANES)]
        for n in range(_NCTX):
            r = b * _NCTX + n
            clo = celo_v[r, pl.ds(0, _LANES)]
            chi = cehi_v[r, pl.ds(0, _LANES)]
            p = clo * wlo + chi * whi
            s = jnp.full((_LANES,), jnp.sum(p))
            plsc.store_scatter(out_v, [jnp.full((_LANES,), r, jnp.int32)],
                               s, mask=last)
        return carry

    lax.fori_loop(0, _BPW, body, 0)

    pltpu.sync_copy(out_v, out_hbm.at[pl.ds(cb, _CPW)])


def kernel(target, context, W_target, W_context):
    tflat = target.reshape(-1)
    cflat = context.reshape(-1)
    wt_lo, wt_hi = _to_banded_rows(W_target.T)
    wc_lo, wc_hi = _to_banded_rows(W_context.T)
    out = _act2vec_sc(tflat, cflat,
                      wt_lo.reshape(_ROWS2, _LANES),
                      wt_hi.reshape(_ROWS2, _LANES),
                      wc_lo.reshape(_ROWS2, _LANES),
                      wc_hi.reshape(_ROWS2, _LANES))
    return out.reshape(_B, _NCTX)


# speedup vs baseline: 1.3864x; 1.0653x over previous
import functools

import jax
import jax.numpy as jnp
from jax import lax
from jax.experimental import pallas as pl
from jax.experimental.pallas import tpu as pltpu
from jax.experimental.pallas import tpu_sc as plsc

_B = 16384
_NCTX = 5
_D = 32
_LANES = 16

_info = plsc.get_sparse_core_info()
_NC, _NS = _info.num_cores, _info.num_subcores
_NW = _NC * _NS
_BPW = _B // _NW
_CPW = _BPW * _NCTX
_CHUNK = 128

_VOCAB = 1000000
_BAND = 126976
_NBAND = 8
_ROWS2 = _BAND * _NBAND
_BNR = 4096
_TGRID = _BAND // _BNR


def _transpose_body(*refs):
    lo_ref, hi_ref = refs[-2], refs[-1]
    eye = jnp.eye(_D, dtype=jnp.float32)
    lo_cols, hi_cols = [], []
    for ref in refs[:-2]:
        t = jax.lax.dot_general(ref[...], eye, (((0,), (0,)), ((), ())),
                                preferred_element_type=jnp.float32)
        lo_cols.append(t[:, :_LANES])
        hi_cols.append(t[:, _LANES:])
    lo_ref[...] = jnp.concatenate(lo_cols, axis=1)
    hi_ref[...] = jnp.concatenate(hi_cols, axis=1)


def _to_banded_rows(w_t):
    last_blk = (_VOCAB - 1) // _BNR
    specs = []
    for qq in range(_NBAND):
        specs.append(pl.BlockSpec(
            (_D, _BNR),
            lambda i, q=qq: (0, jnp.minimum(q * _TGRID + i, last_blk))))
    return pl.pallas_call(
        _transpose_body,
        grid=(_TGRID,),
        in_specs=specs,
        out_specs=[pl.BlockSpec((_BNR, 128), lambda i: (i, 0)),
                   pl.BlockSpec((_BNR, 128), lambda i: (i, 0))],
        out_shape=[jax.ShapeDtypeStruct((_BAND, 128), jnp.float32),
                   jax.ShapeDtypeStruct((_BAND, 128), jnp.float32)],
    )(*([w_t] * _NBAND))


_mesh = plsc.VectorSubcoreMesh(core_axis_name="c", subcore_axis_name="s")


@functools.partial(
    pl.kernel,
    mesh=_mesh,
    out_type=jax.ShapeDtypeStruct((_B * _NCTX,), jnp.float32),
    scratch_types=[
        pltpu.VMEM((_BPW,), jnp.int32),
        pltpu.VMEM((_CPW,), jnp.int32),
        pltpu.VMEM((_BPW, _LANES), jnp.float32),
        pltpu.VMEM((_BPW, _LANES), jnp.float32),
        pltpu.VMEM((_CPW, _LANES), jnp.float32),
        pltpu.VMEM((_CPW, _LANES), jnp.float32),
        pltpu.VMEM((_CPW,), jnp.float32),
        pltpu.SemaphoreType.DMA,
    ],
    compiler_params=pltpu.CompilerParams(
        needs_layout_passes=False, use_tc_tiling_on_sc=False),
)
def _act2vec_sc(t_hbm, c_hbm, wtlo_hbm, wthi_hbm, wclo_hbm, wchi_hbm, out_hbm,
                tix_v, cix_v, welo_v, wehi_v, celo_v, cehi_v, out_v, sem):
    wid = lax.axis_index("s") * _NC + lax.axis_index("c")
    tb = wid * _BPW
    cb = wid * _CPW

    pltpu.sync_copy(t_hbm.at[pl.ds(tb, _BPW)], tix_v)
    pltpu.sync_copy(c_hbm.at[pl.ds(cb, _CPW)], cix_v)

    def band_fix(ref, nvec):
        def vfix(i, carry):
            r = ref[pl.ds(i * _LANES, _LANES)]
            q = (r >= _BAND).astype(jnp.int32)
            for k in range(2, _NBAND):
                q = q + (r >= k * _BAND).astype(jnp.int32)
            ref[pl.ds(i * _LANES, _LANES)] = (r - q * _BAND) * _NBAND + q
            return carry
        lax.fori_loop(0, nvec, vfix, 0)

    band_fix(tix_v, _BPW // _LANES)
    band_fix(cix_v, _CPW // _LANES)

    copies = []
    for j in range(_BPW // _CHUNK):
        ds = pl.ds(j * _CHUNK, _CHUNK)
        copies.append(pltpu.async_copy(wtlo_hbm.at[tix_v.at[ds]],
                                       welo_v.at[ds], sem))
        copies.append(pltpu.async_copy(wthi_hbm.at[tix_v.at[ds]],
                                       wehi_v.at[ds], sem))
    for j in range(_CPW // _CHUNK):
        ds = pl.ds(j * _CHUNK, _CHUNK)
        copies.append(pltpu.async_copy(wclo_hbm.at[cix_v.at[ds]],
                                       celo_v.at[ds], sem))
        copies.append(pltpu.async_copy(wchi_hbm.at[cix_v.at[ds]],
                                       cehi_v.at[ds], sem))
    for c in copies:
        c.wait()

    last = lax.iota(jnp.int32, _LANES) == (_LANES - 1)

    def body(b, carry):
        wlo = welo_v[b, pl.ds(0, _LANES)]
        whi = wehi_v[b, pl.ds(0, _LANES)]
        for n in range(_NCTX):
            r = b * _NCTX + n
            clo = celo_v[r, pl.ds(0, _LANES)]
            chi = cehi_v[r, pl.ds(0, _LANES)]
            p = clo * wlo + chi * whi
            s = jnp.full((_LANES,), jnp.sum(p))
            plsc.store_scatter(out_v, [jnp.full((_LANES,), r, jnp.int32)],
                               s, mask=last)
        return carry

    lax.fori_loop(0, _BPW, body, 0)

    pltpu.sync_copy(out_v, out_hbm.at[pl.ds(cb, _CPW)])


def kernel(target, context, W_target, W_context):
    tflat = target.reshape(-1)
    cflat = context.reshape(-1)
    wt_lo, wt_hi = _to_banded_rows(W_target.T)
    wc_lo, wc_hi = _to_banded_rows(W_context.T)
    out = _act2vec_sc(tflat, cflat,
                      wt_lo.reshape(_ROWS2, _LANES),
                      wt_hi.reshape(_ROWS2, _LANES),
                      wc_lo.reshape(_ROWS2, _LANES),
                      wc_hi.reshape(_ROWS2, _LANES))
    return out.reshape(_B, _NCTX)
